# baseline (speedup 1.0000x reference)
import jax
import jax.numpy as jnp
from jax import lax
from jax.experimental import pallas as pl
from jax.experimental.pallas import tpu as pltpu
from jax.experimental.pallas import tpu_sc as plsc

N_ENV = 512
BUF = 1024
N_OBS = 64
N_ACT = 16
BATCH = 256

NC = 2
NS = 16
NW = NC * NS
L = 16
EPW = N_ENV // NW
NCHUNK = 2
CH = BATCH // NCHUNK
NGC = CH // L
NG = BATCH // L


def _worker_id():
    return lax.axis_index("s") * NC + lax.axis_index("c")


def _body(obs128, nobs128, act128, rew_buf, don_buf, ter_buf, tou_buf,
          obs_new, nobs_new, act_new, rew_new, don_new, ter_new, tou_new,
          idx3, p_arr,
          obs_out, nobs_out, act_out, rew_out, don_out, ter_out, tou_out,
          ens_out,
          idx_v, gidx2, gidx8,
          wide, nwide, awide,
          obs_rows_a, obs_rows_b, nobs_rows_a, nobs_rows_b,
          act_rows_a, act_rows_b,
          rew_row, don_row, ter_row, tou_row,
          rew_so, don_so, ter_so, tou_so, ens_so,
          obs_ne, nobs_ne, act_ne,
          rew16, don16, ter16, tou16, p_v,
          sem_g):
    w = _worker_id()
    base_env = w * EPW

    pltpu.sync_copy(p_arr, p_v)
    pltpu.sync_copy(rew_new.at[pl.ds(base_env, EPW)], rew16.at[pl.ds(0, EPW)])
    pltpu.sync_copy(don_new.at[pl.ds(base_env, EPW)], don16.at[pl.ds(0, EPW)])
    pltpu.sync_copy(ter_new.at[pl.ds(base_env, EPW)], ter16.at[pl.ds(0, EPW)])
    pltpu.sync_copy(tou_new.at[pl.ds(base_env, EPW)], tou16.at[pl.ds(0, EPW)])
    pv = p_v[...]
    ones16 = jnp.ones((L,), jnp.int32)
    for g in range(NG):
        ens_so[pl.ds(g * L, L)] = ones16

    def env_body(j, carry):
        e = base_env + j
        pltpu.sync_copy(idx3.at[e], idx_v)

        ebase2 = e * (BUF // 2)
        ebase8 = e * (BUF // 8)
        hit_acc = jnp.zeros((L,), jnp.bool_)
        for g in range(NG):
            c, o = g // NGC, (g % NGC) * L
            iv = idx_v[c, pl.ds(o, L)]
            gidx2[c][pl.ds(o, L)] = lax.shift_right_logical(iv, 1) + ebase2
            gidx8[c][pl.ds(o, L)] = lax.shift_right_logical(iv, 3) + ebase8
            hit_acc = jnp.logical_or(hit_acc, iv == pv)
        anyhit = plsc.all_reduce_population_count(hit_acc)[0] > 0

        rew_e = jnp.full((L,), rew16[pl.ds(j, L)][0])
        don_e = jnp.full((L,), don16[pl.ds(j, L)][0])
        ter_e = jnp.full((L,), ter16[pl.ds(j, L)][0])
        tou_e = jnp.full((L,), tou16[pl.ds(j, L)][0])

        obs_rows = (obs_rows_a, obs_rows_b)
        nobs_rows = (nobs_rows_a, nobs_rows_b)
        act_rows = (act_rows_a, act_rows_b)

        def fire(c):
            return [
                pltpu.async_copy(obs128.at[gidx2[c]], wide, sem_g),
                pltpu.async_copy(nobs128.at[gidx2[c]], nwide, sem_g),
                pltpu.async_copy(act128.at[gidx8[c]], awide, sem_g),
            ]

        def compact(c):
            for g in range(NGC):
                o = g * L
                iv = idx_v[c, pl.ds(o, L)]
                rows = lax.iota(jnp.int32, L) + o
                col2 = (iv & 1) * N_OBS
                col8 = (iv & 7) * N_ACT

                def fbody(f, carry, c=c, rows=rows, col2=col2):
                    fv = jnp.full((L,), f, jnp.int32)
                    vals = plsc.load_gather(wide, [rows, col2 + f])
                    plsc.store_scatter(obs_rows[c], [rows, fv], vals)
                    nvals = plsc.load_gather(nwide, [rows, col2 + f])
                    plsc.store_scatter(nobs_rows[c], [rows, fv], nvals)
                    return carry

                lax.fori_loop(0, N_OBS, fbody, 0, unroll=4)

                def abody(f, carry, c=c, rows=rows, col8=col8):
                    fv = jnp.full((L,), f, jnp.int32)
                    avals = plsc.load_gather(awide, [rows, col8 + f])
                    plsc.store_scatter(act_rows[c], [rows, fv], avals)
                    return carry

                lax.fori_loop(0, N_ACT, abody, 0, unroll=4)

        copies = fire(0)

        pltpu.sync_copy(rew_buf.at[e], rew_row)
        pltpu.sync_copy(don_buf.at[e], don_row)
        pltpu.sync_copy(ter_buf.at[e], ter_row)
        pltpu.sync_copy(tou_buf.at[e], tou_row)

        for g in range(NG):
            iv = idx_v[g // NGC, pl.ds((g % NGC) * L, L)]
            m = iv == pv
            rew_so[pl.ds(g * L, L)] = jnp.where(
                m, rew_e, plsc.load_gather(rew_row, [iv]))
            don_so[pl.ds(g * L, L)] = jnp.where(
                m, don_e, plsc.load_gather(don_row, [iv]))
            ter_so[pl.ds(g * L, L)] = jnp.where(
                m, ter_e, plsc.load_gather(ter_row, [iv]))
            tou_so[pl.ds(g * L, L)] = jnp.where(
                m, tou_e, plsc.load_gather(tou_row, [iv]))

        for cp in copies:
            cp.wait()
        compact(0)
        copies = fire(1)
        for cp in copies:
            cp.wait()
        compact(1)

        @pl.when(anyhit)
        def _fix():
            pltpu.sync_copy(obs_new.at[e], obs_ne)
            pltpu.sync_copy(nobs_new.at[e], nobs_ne)
            pltpu.sync_copy(act_new.at[e], act_ne)
            onew = [obs_ne[pl.ds(k * L, L)] for k in range(N_OBS // L)]
            nnew = [nobs_ne[pl.ds(k * L, L)] for k in range(N_OBS // L)]
            anew = act_ne[...]
            for g in range(NG):
                iv = idx_v[g // NGC, pl.ds((g % NGC) * L, L)]
                m = iv == pv
                mi = jnp.where(m, 1, 0).astype(jnp.int32)

                @pl.when(plsc.all_reduce_population_count(m)[0] > 0)
                def _fix_group(g=g, mi=mi):
                    ck = g // NGC
                    for lane in range(L):
                        @pl.when(mi[lane] != 0)
                        def _fix_lane(g=g, lane=lane, ck=ck):
                            b = (g % NGC) * L + lane
                            for k in range(N_OBS // L):
                                obs_rows[ck][b, pl.ds(k * L, L)] = onew[k]
                                nobs_rows[ck][b, pl.ds(k * L, L)] = nnew[k]
                            act_rows[ck][b, :] = anew

        ob = e * BATCH
        for c in range(NCHUNK):
            pltpu.sync_copy(obs_rows[c], obs_out.at[pl.ds(ob + c * CH, CH)])
            pltpu.sync_copy(nobs_rows[c], nobs_out.at[pl.ds(ob + c * CH, CH)])
            pltpu.sync_copy(act_rows[c], act_out.at[pl.ds(ob + c * CH, CH)])
        pltpu.sync_copy(rew_so, rew_out.at[pl.ds(ob, BATCH)])
        pltpu.sync_copy(don_so, don_out.at[pl.ds(ob, BATCH)])
        pltpu.sync_copy(ter_so, ter_out.at[pl.ds(ob, BATCH)])
        pltpu.sync_copy(tou_so, tou_out.at[pl.ds(ob, BATCH)])
        pltpu.sync_copy(ens_so, ens_out.at[pl.ds(ob, BATCH)])
        return carry

    lax.fori_loop(0, EPW, env_body, 0)


_OUT_TYPE = (
    jax.ShapeDtypeStruct((N_ENV * BATCH, N_OBS), jnp.float32),
    jax.ShapeDtypeStruct((N_ENV * BATCH, N_OBS), jnp.float32),
    jax.ShapeDtypeStruct((N_ENV * BATCH, N_ACT), jnp.float32),
    jax.ShapeDtypeStruct((N_ENV * BATCH,), jnp.float32),
    jax.ShapeDtypeStruct((N_ENV * BATCH,), jnp.int32),
    jax.ShapeDtypeStruct((N_ENV * BATCH,), jnp.int32),
    jax.ShapeDtypeStruct((N_ENV * BATCH,), jnp.int32),
    jax.ShapeDtypeStruct((N_ENV * BATCH,), jnp.int32),
)

_SCRATCH = [
    pltpu.VMEM((NCHUNK, CH), jnp.int32),
    [pltpu.VMEM((CH,), jnp.int32)] * NCHUNK,
    [pltpu.VMEM((CH,), jnp.int32)] * NCHUNK,
    pltpu.VMEM((CH, 128), jnp.float32),
    pltpu.VMEM((CH, 128), jnp.float32),
    pltpu.VMEM((CH, 128), jnp.float32),
    pltpu.VMEM((CH, N_OBS), jnp.float32),
    pltpu.VMEM((CH, N_OBS), jnp.float32),
    pltpu.VMEM((CH, N_OBS), jnp.float32),
    pltpu.VMEM((CH, N_OBS), jnp.float32),
    pltpu.VMEM((CH, N_ACT), jnp.float32),
    pltpu.VMEM((CH, N_ACT), jnp.float32),
    pltpu.VMEM((BUF,), jnp.float32),
    pltpu.VMEM((BUF,), jnp.int32),
    pltpu.VMEM((BUF,), jnp.int32),
    pltpu.VMEM((BUF,), jnp.int32),
    pltpu.VMEM((BATCH,), jnp.float32),
    pltpu.VMEM((BATCH,), jnp.int32),
    pltpu.VMEM((BATCH,), jnp.int32),
    pltpu.VMEM((BATCH,), jnp.int32),
    pltpu.VMEM((BATCH,), jnp.int32),
    pltpu.VMEM((N_OBS,), jnp.float32),
    pltpu.VMEM((N_OBS,), jnp.float32),
    pltpu.VMEM((N_ACT,), jnp.float32),
    pltpu.VMEM((EPW + L,), jnp.float32),
    pltpu.VMEM((EPW + L,), jnp.int32),
    pltpu.VMEM((EPW + L,), jnp.int32),
    pltpu.VMEM((EPW + L,), jnp.int32),
    pltpu.VMEM((L,), jnp.int32),
    pltpu.SemaphoreType.DMA,
]

_sc_call = pl.kernel(
    _body,
    out_type=_OUT_TYPE,
    mesh=plsc.VectorSubcoreMesh(core_axis_name="c", subcore_axis_name="s",
                                num_cores=NC, num_subcores=NS),
    scratch_types=_SCRATCH,
    compiler_params=pltpu.CompilerParams(needs_layout_passes=False,
                                         use_tc_tiling_on_sc=False),
)


def kernel(observations_buf, next_observations_buf, actions_buf, rewards_buf,
           dones_buf, terminations_buf, time_outs_buf,
           obs, actions_in, rewards_in, next_obs,
           dones_in, terminations_in, time_outs_in,
           indices, ptr):
    p = jnp.asarray(ptr, jnp.int32) % BUF
    p_arr = jnp.full((L,), p, jnp.int32)
    obs_flat = observations_buf.reshape(N_ENV * BUF * N_OBS // 128, 128)
    nobs_flat = next_observations_buf.reshape(N_ENV * BUF * N_OBS // 128, 128)
    act_flat = actions_buf.reshape(N_ENV * BUF * N_ACT // 128, 128)
    idx3 = indices.reshape(N_ENV, NCHUNK, CH)
    return _sc_call(
        obs_flat, nobs_flat, act_flat, rewards_buf, dones_buf,
        terminations_buf, time_outs_buf,
        obs, next_obs, actions_in, rewards_in,
        dones_in, terminations_in, time_outs_in,
        idx3, p_arr)


# speedup vs baseline: 1.4115x; 1.4115x over previous
import jax
import jax.numpy as jnp
from jax import lax
from jax.experimental import pallas as pl
from jax.experimental.pallas import tpu as pltpu
from jax.experimental.pallas import tpu_sc as plsc

N_ENV = 512
BUF = 1024
N_OBS = 64
N_ACT = 16
BATCH = 256

NC = 2
NS = 16
NW = NC * NS
L = 16
EPW = N_ENV // NW
NCHUNK = 2
CH = BATCH // NCHUNK
NGC = CH // L
NG = BATCH // L

OBS_PR = 128 // N_OBS
ACT_PR = 128 // N_ACT


def _worker_id():
    return lax.axis_index("s") * NC + lax.axis_index("c")


def _body(obs128, nobs128, act128, rew_buf, don_buf, ter_buf, tou_buf,
          obs_new, nobs_new, act_new, rew_new, don_new, ter_new, tou_new,
          idx1, p_arr,
          obs_out, nobs_out, act_out, rew_out, don_out, ter_out, tou_out,
          ens_out,
          idx_v, gidx2, gidx8, pcol2, pcol8,
          wide, nwide, awide,
          obs_st, nobs_st, act_st,
          rew_row, don_row, ter_row, tou_row,
          rew_so, don_so, ter_so, tou_so, ens_so,
          obs_ne, nobs_ne, act_ne,
          rew16, don16, ter16, tou16, p_v,
          sem_g):
    w = _worker_id()
    base_env = w * EPW

    pltpu.sync_copy(p_arr, p_v)
    pltpu.sync_copy(rew_new.at[pl.ds(base_env, EPW)], rew16.at[pl.ds(0, EPW)])
    pltpu.sync_copy(don_new.at[pl.ds(base_env, EPW)], don16.at[pl.ds(0, EPW)])
    pltpu.sync_copy(ter_new.at[pl.ds(base_env, EPW)], ter16.at[pl.ds(0, EPW)])
    pltpu.sync_copy(tou_new.at[pl.ds(base_env, EPW)], tou16.at[pl.ds(0, EPW)])
    pv = p_v[...]
    ones16 = jnp.ones((L,), jnp.int32)
    for g in range(NG):
        ens_so[pl.ds(g * L, L)] = ones16

    def env_body(j, carry):
        e = base_env + j
        pltpu.sync_copy(idx1.at[pl.ds(e * BATCH, BATCH)], idx_v)

        ebase2 = e * (BUF // OBS_PR)
        ebase8 = e * (BUF // ACT_PR)
        hit_acc = jnp.zeros((L,), jnp.bool_)
        for g in range(NG):
            c, o = g // NGC, (g % NGC) * L
            iv = idx_v[pl.ds(c * CH + o, L)]
            gidx2[c][pl.ds(o, L)] = lax.shift_right_logical(iv, 1) + ebase2
            gidx8[c][pl.ds(o, L)] = lax.shift_right_logical(iv, 3) + ebase8
            pcol2[pl.ds(g * L, L)] = (iv & (OBS_PR - 1)) * N_OBS
            pcol8[pl.ds(g * L, L)] = (iv & (ACT_PR - 1)) * N_ACT
            hit_acc = jnp.logical_or(hit_acc, iv == pv)
        anyhit = plsc.all_reduce_population_count(hit_acc)[0] > 0

        rew_e = jnp.full((L,), rew16[pl.ds(j, L)][0])
        don_e = jnp.full((L,), don16[pl.ds(j, L)][0])
        ter_e = jnp.full((L,), ter16[pl.ds(j, L)][0])
        tou_e = jnp.full((L,), tou16[pl.ds(j, L)][0])

        def fire(c):
            return [
                pltpu.async_copy(obs128.at[gidx2[c]], wide[c], sem_g),
                pltpu.async_copy(nobs128.at[gidx2[c]], nwide[c], sem_g),
                pltpu.async_copy(act128.at[gidx8[c]], awide[c], sem_g),
            ]

        def compact_and_flush(c):
            def sbody(b, carry):
                h = pcol2[pl.ds(c * CH + b, L)][0]
                h8 = pcol8[pl.ds(c * CH + b, L)][0]
                r2 = b // OBS_PR
                c2 = (b % OBS_PR) * N_OBS
                for k in range(N_OBS // L):
                    obs_st[r2, pl.ds(c2 + k * L, L)] = (
                        wide[c][b, pl.ds(h + k * L, L)])
                    nobs_st[r2, pl.ds(c2 + k * L, L)] = (
                        nwide[c][b, pl.ds(h + k * L, L)])
                act_st[b // ACT_PR, pl.ds((b % ACT_PR) * N_ACT, L)] = (
                    awide[c][b, pl.ds(h8, L)])
                return carry

            lax.fori_loop(0, CH, sbody, 0, unroll=2)

        copies = fire(0)

        pltpu.sync_copy(rew_buf.at[pl.ds(e * BUF, BUF)], rew_row)
        pltpu.sync_copy(don_buf.at[pl.ds(e * BUF, BUF)], don_row)
        pltpu.sync_copy(ter_buf.at[pl.ds(e * BUF, BUF)], ter_row)
        pltpu.sync_copy(tou_buf.at[pl.ds(e * BUF, BUF)], tou_row)

        for g in range(NG):
            iv = idx_v[pl.ds(g * L, L)]
            m = iv == pv
            rew_so[pl.ds(g * L, L)] = jnp.where(
                m, rew_e, plsc.load_gather(rew_row, [iv]))
            don_so[pl.ds(g * L, L)] = jnp.where(
                m, don_e, plsc.load_gather(don_row, [iv]))
            ter_so[pl.ds(g * L, L)] = jnp.where(
                m, ter_e, plsc.load_gather(ter_row, [iv]))
            tou_so[pl.ds(g * L, L)] = jnp.where(
                m, tou_e, plsc.load_gather(tou_row, [iv]))

        def fix_chunk(c):
            for g in range(NGC):
                iv = idx_v[pl.ds(c * CH + g * L, L)]
                m = iv == pv
                mi = jnp.where(m, 1, 0).astype(jnp.int32)

                @pl.when(plsc.all_reduce_population_count(m)[0] > 0)
                def _fix_group(g=g, mi=mi, c=c):
                    onew = [obs_ne[pl.ds(k * L, L)]
                            for k in range(N_OBS // L)]
                    nnew = [nobs_ne[pl.ds(k * L, L)]
                            for k in range(N_OBS // L)]
                    anew = act_ne[...]
                    for lane in range(L):
                        @pl.when(mi[lane] != 0)
                        def _fix_lane(g=g, lane=lane):
                            b = g * L + lane
                            r2 = b // OBS_PR
                            c2 = (b % OBS_PR) * N_OBS
                            for k in range(N_OBS // L):
                                obs_st[r2, pl.ds(c2 + k * L, L)] = onew[k]
                                nobs_st[r2, pl.ds(c2 + k * L, L)] = nnew[k]
                            act_st[b // ACT_PR,
                                   pl.ds((b % ACT_PR) * N_ACT, L)] = anew

        @pl.when(anyhit)
        def _load_new():
            pltpu.sync_copy(obs_new.at[pl.ds(e * N_OBS, N_OBS)], obs_ne)
            pltpu.sync_copy(nobs_new.at[pl.ds(e * N_OBS, N_OBS)], nobs_ne)
            pltpu.sync_copy(act_new.at[pl.ds(e * N_ACT, N_ACT)], act_ne)

        out_copies = []
        for c in range(NCHUNK):
            for cp in copies:
                cp.wait()
            if c + 1 < NCHUNK:
                copies = fire(c + 1)
            compact_and_flush(c)

            @pl.when(anyhit)
            def _fix(c=c):
                fix_chunk(c)

            ob2 = e * (BATCH // OBS_PR) + c * (CH // OBS_PR)
            ob8 = e * (BATCH // ACT_PR) + c * (CH // ACT_PR)
            out_copies.append(pltpu.async_copy(
                obs_st, obs_out.at[pl.ds(ob2, CH // OBS_PR)], sem_g))
            out_copies.append(pltpu.async_copy(
                nobs_st, nobs_out.at[pl.ds(ob2, CH // OBS_PR)], sem_g))
            out_copies.append(pltpu.async_copy(
                act_st, act_out.at[pl.ds(ob8, CH // ACT_PR)], sem_g))
            if c + 1 < NCHUNK:
                for cp in out_copies:
                    cp.wait()
                out_copies = []

        ob = e * BATCH
        out_copies.append(pltpu.async_copy(
            rew_so, rew_out.at[pl.ds(ob, BATCH)], sem_g))
        out_copies.append(pltpu.async_copy(
            don_so, don_out.at[pl.ds(ob, BATCH)], sem_g))
        out_copies.append(pltpu.async_copy(
            ter_so, ter_out.at[pl.ds(ob, BATCH)], sem_g))
        out_copies.append(pltpu.async_copy(
            tou_so, tou_out.at[pl.ds(ob, BATCH)], sem_g))
        out_copies.append(pltpu.async_copy(
            ens_so, ens_out.at[pl.ds(ob, BATCH)], sem_g))
        for cp in out_copies:
            cp.wait()
        return carry

    lax.fori_loop(0, EPW, env_body, 0)


_OUT_TYPE = (
    jax.ShapeDtypeStruct((N_ENV * BATCH // OBS_PR, 128), jnp.float32),
    jax.ShapeDtypeStruct((N_ENV * BATCH // OBS_PR, 128), jnp.float32),
    jax.ShapeDtypeStruct((N_ENV * BATCH // ACT_PR, 128), jnp.float32),
    jax.ShapeDtypeStruct((N_ENV * BATCH,), jnp.float32),
    jax.ShapeDtypeStruct((N_ENV * BATCH,), jnp.int32),
    jax.ShapeDtypeStruct((N_ENV * BATCH,), jnp.int32),
    jax.ShapeDtypeStruct((N_ENV * BATCH,), jnp.int32),
    jax.ShapeDtypeStruct((N_ENV * BATCH,), jnp.int32),
)

_SCRATCH = [
    pltpu.VMEM((BATCH,), jnp.int32),
    [pltpu.VMEM((CH,), jnp.int32)] * NCHUNK,
    [pltpu.VMEM((CH,), jnp.int32)] * NCHUNK,
    pltpu.VMEM((BATCH + L,), jnp.int32),
    pltpu.VMEM((BATCH + L,), jnp.int32),
    [pltpu.VMEM((CH, 128), jnp.float32)] * NCHUNK,
    [pltpu.VMEM((CH, 128), jnp.float32)] * NCHUNK,
    [pltpu.VMEM((CH, 128), jnp.float32)] * NCHUNK,
    pltpu.VMEM((CH // OBS_PR, 128), jnp.float32),
    pltpu.VMEM((CH // OBS_PR, 128), jnp.float32),
    pltpu.VMEM((CH // ACT_PR, 128), jnp.float32),
    pltpu.VMEM((BUF,), jnp.float32),
    pltpu.VMEM((BUF,), jnp.int32),
    pltpu.VMEM((BUF,), jnp.int32),
    pltpu.VMEM((BUF,), jnp.int32),
    pltpu.VMEM((BATCH,), jnp.float32),
    pltpu.VMEM((BATCH,), jnp.int32),
    pltpu.VMEM((BATCH,), jnp.int32),
    pltpu.VMEM((BATCH,), jnp.int32),
    pltpu.VMEM((BATCH,), jnp.int32),
    pltpu.VMEM((N_OBS,), jnp.float32),
    pltpu.VMEM((N_OBS,), jnp.float32),
    pltpu.VMEM((N_ACT,), jnp.float32),
    pltpu.VMEM((EPW + L,), jnp.float32),
    pltpu.VMEM((EPW + L,), jnp.int32),
    pltpu.VMEM((EPW + L,), jnp.int32),
    pltpu.VMEM((EPW + L,), jnp.int32),
    pltpu.VMEM((L,), jnp.int32),
    pltpu.SemaphoreType.DMA,
]

_sc_call = pl.kernel(
    _body,
    out_type=_OUT_TYPE,
    mesh=plsc.VectorSubcoreMesh(core_axis_name="c", subcore_axis_name="s",
                                num_cores=NC, num_subcores=NS),
    scratch_types=_SCRATCH,
    compiler_params=pltpu.CompilerParams(needs_layout_passes=False),
)


def kernel(observations_buf, next_observations_buf, actions_buf, rewards_buf,
           dones_buf, terminations_buf, time_outs_buf,
           obs, actions_in, rewards_in, next_obs,
           dones_in, terminations_in, time_outs_in,
           indices, ptr):
    p = jnp.asarray(ptr, jnp.int32) % BUF
    p_arr = jnp.full((L,), p, jnp.int32)
    obs_flat = observations_buf.reshape(N_ENV * BUF * N_OBS // 128, 128)
    nobs_flat = next_observations_buf.reshape(N_ENV * BUF * N_OBS // 128, 128)
    act_flat = actions_buf.reshape(N_ENV * BUF * N_ACT // 128, 128)
    idx1 = indices.reshape(-1)
    (obs128_o, nobs128_o, act128_o, rewards, dones, terminations, time_outs,
     ens) = _sc_call(
        obs_flat, nobs_flat, act_flat,
        rewards_buf.reshape(-1), dones_buf.reshape(-1),
        terminations_buf.reshape(-1), time_outs_buf.reshape(-1),
        obs.reshape(-1), next_obs.reshape(-1), actions_in.reshape(-1),
        rewards_in, dones_in, terminations_in, time_outs_in,
        idx1, p_arr)
    observations = obs128_o.reshape(N_ENV * BATCH, N_OBS)
    next_observations = nobs128_o.reshape(N_ENV * BATCH, N_OBS)
    actions = act128_o.reshape(N_ENV * BATCH, N_ACT)
    return (observations, next_observations, actions, rewards, dones,
            terminations, time_outs, ens)


# speedup vs baseline: 1.4120x; 1.0004x over previous
import jax
import jax.numpy as jnp
from jax import lax
from jax.experimental import pallas as pl
from jax.experimental.pallas import tpu as pltpu
from jax.experimental.pallas import tpu_sc as plsc

N_ENV = 512
BUF = 1024
N_OBS = 64
N_ACT = 16
BATCH = 256

NC = 2
NS = 16
NW = NC * NS
L = 16
EPW = N_ENV // NW
NCHUNK = 2
CH = BATCH // NCHUNK
NGC = CH // L
NG = BATCH // L

OBS_PR = 128 // N_OBS
ACT_PR = 128 // N_ACT


def _worker_id():
    return lax.axis_index("s") * NC + lax.axis_index("c")


def _body(obs128, nobs128, act128, rew_buf, don_buf, ter_buf, tou_buf,
          obs_new, nobs_new, act_new, rew_new, don_new, ter_new, tou_new,
          idx1, p_arr,
          obs_out, nobs_out, act_out, rew_out, don_out, ter_out, tou_out,
          ens_out,
          idx_v, gidx2, gidx8, pcol2, pcol8,
          wide, nwide, awide,
          obs_st, nobs_st, act_st,
          rew_row, don_row, ter_row, tou_row,
          rew_so, don_so, ter_so, tou_so, ens_so,
          obs_ne, nobs_ne, act_ne,
          rew16, don16, ter16, tou16, p_v,
          sem_g):
    w = _worker_id()
    base_env = w * EPW

    pltpu.sync_copy(p_arr, p_v)
    pltpu.sync_copy(rew_new.at[pl.ds(base_env, EPW)], rew16.at[pl.ds(0, EPW)])
    pltpu.sync_copy(don_new.at[pl.ds(base_env, EPW)], don16.at[pl.ds(0, EPW)])
    pltpu.sync_copy(ter_new.at[pl.ds(base_env, EPW)], ter16.at[pl.ds(0, EPW)])
    pltpu.sync_copy(tou_new.at[pl.ds(base_env, EPW)], tou16.at[pl.ds(0, EPW)])
    pv = p_v[...]
    ones16 = jnp.ones((L,), jnp.int32)
    for g in range(NG):
        ens_so[pl.ds(g * L, L)] = ones16

    def env_body(j, carry):
        e = base_env + j
        pltpu.sync_copy(idx1.at[pl.ds(e * BATCH, BATCH)], idx_v)

        ebase2 = e * (BUF // OBS_PR)
        ebase8 = e * (BUF // ACT_PR)
        hit_acc = jnp.zeros((L,), jnp.bool_)
        for g in range(NG):
            c, o = g // NGC, (g % NGC) * L
            iv = idx_v[pl.ds(c * CH + o, L)]
            gidx2[c][pl.ds(o, L)] = lax.shift_right_logical(iv, 1) + ebase2
            gidx8[c][pl.ds(o, L)] = lax.shift_right_logical(iv, 3) + ebase8
            pcol2[pl.ds(g * L, L)] = (iv & (OBS_PR - 1)) * N_OBS
            pcol8[pl.ds(g * L, L)] = (iv & (ACT_PR - 1)) * N_ACT
            hit_acc = jnp.logical_or(hit_acc, iv == pv)
        anyhit = plsc.all_reduce_population_count(hit_acc)[0] > 0

        rew_e = jnp.full((L,), rew16[pl.ds(j, L)][0])
        don_e = jnp.full((L,), don16[pl.ds(j, L)][0])
        ter_e = jnp.full((L,), ter16[pl.ds(j, L)][0])
        tou_e = jnp.full((L,), tou16[pl.ds(j, L)][0])

        def fire(c):
            return [
                pltpu.async_copy(obs128.at[gidx2[c]], wide[c], sem_g),
                pltpu.async_copy(nobs128.at[gidx2[c]], nwide[c], sem_g),
                pltpu.async_copy(act128.at[gidx8[c]], awide[c], sem_g),
            ]

        def compact_and_flush(c):
            def sbody(b, carry):
                h = pcol2[pl.ds(c * CH + b, L)][0]
                h8 = pcol8[pl.ds(c * CH + b, L)][0]
                r2 = b // OBS_PR
                c2 = (b % OBS_PR) * N_OBS
                for k in range(N_OBS // L):
                    obs_st[r2, pl.ds(c2 + k * L, L)] = (
                        wide[c][b, pl.ds(h + k * L, L)])
                    nobs_st[r2, pl.ds(c2 + k * L, L)] = (
                        nwide[c][b, pl.ds(h + k * L, L)])
                act_st[b // ACT_PR, pl.ds((b % ACT_PR) * N_ACT, L)] = (
                    awide[c][b, pl.ds(h8, L)])
                return carry

            lax.fori_loop(0, CH, sbody, 0, unroll=2)

        copies = fire(0)

        pltpu.sync_copy(rew_buf.at[pl.ds(e * BUF, BUF)], rew_row)
        pltpu.sync_copy(don_buf.at[pl.ds(e * BUF, BUF)], don_row)
        pltpu.sync_copy(ter_buf.at[pl.ds(e * BUF, BUF)], ter_row)
        pltpu.sync_copy(tou_buf.at[pl.ds(e * BUF, BUF)], tou_row)

        for g in range(NG):
            iv = idx_v[pl.ds(g * L, L)]
            m = iv == pv
            rew_so[pl.ds(g * L, L)] = jnp.where(
                m, rew_e, plsc.load_gather(rew_row, [iv]))
            don_so[pl.ds(g * L, L)] = jnp.where(
                m, don_e, plsc.load_gather(don_row, [iv]))
            ter_so[pl.ds(g * L, L)] = jnp.where(
                m, ter_e, plsc.load_gather(ter_row, [iv]))
            tou_so[pl.ds(g * L, L)] = jnp.where(
                m, tou_e, plsc.load_gather(tou_row, [iv]))

        def fix_chunk(c):
            for g in range(NGC):
                iv = idx_v[pl.ds(c * CH + g * L, L)]
                m = iv == pv
                mi = jnp.where(m, 1, 0).astype(jnp.int32)

                @pl.when(plsc.all_reduce_population_count(m)[0] > 0)
                def _fix_group(g=g, mi=mi, c=c):
                    onew = [obs_ne[pl.ds(k * L, L)]
                            for k in range(N_OBS // L)]
                    nnew = [nobs_ne[pl.ds(k * L, L)]
                            for k in range(N_OBS // L)]
                    anew = act_ne[...]
                    for lane in range(L):
                        @pl.when(mi[lane] != 0)
                        def _fix_lane(g=g, lane=lane):
                            b = g * L + lane
                            r2 = b // OBS_PR
                            c2 = (b % OBS_PR) * N_OBS
                            for k in range(N_OBS // L):
                                obs_st[r2, pl.ds(c2 + k * L, L)] = onew[k]
                                nobs_st[r2, pl.ds(c2 + k * L, L)] = nnew[k]
                            act_st[b // ACT_PR,
                                   pl.ds((b % ACT_PR) * N_ACT, L)] = anew

        @pl.when(anyhit)
        def _load_new():
            pltpu.sync_copy(obs_new.at[pl.ds(e * N_OBS, N_OBS)], obs_ne)
            pltpu.sync_copy(nobs_new.at[pl.ds(e * N_OBS, N_OBS)], nobs_ne)
            pltpu.sync_copy(act_new.at[pl.ds(e * N_ACT, N_ACT)], act_ne)

        out_copies = []
        for c in range(NCHUNK):
            for cp in copies:
                cp.wait()
            if c + 1 < NCHUNK:
                copies = fire(c + 1)
            compact_and_flush(c)

            @pl.when(anyhit)
            def _fix(c=c):
                fix_chunk(c)

            ob2 = e * (BATCH // OBS_PR) + c * (CH // OBS_PR)
            ob8 = e * (BATCH // ACT_PR) + c * (CH // ACT_PR)
            out_copies.append(pltpu.async_copy(
                obs_st, obs_out.at[pl.ds(ob2, CH // OBS_PR)], sem_g))
            out_copies.append(pltpu.async_copy(
                nobs_st, nobs_out.at[pl.ds(ob2, CH // OBS_PR)], sem_g))
            out_copies.append(pltpu.async_copy(
                act_st, act_out.at[pl.ds(ob8, CH // ACT_PR)], sem_g))
            if c + 1 < NCHUNK:
                for cp in out_copies:
                    cp.wait()
                out_copies = []

        ob = e * BATCH
        out_copies.append(pltpu.async_copy(
            rew_so, rew_out.at[pl.ds(ob, BATCH)], sem_g))
        out_copies.append(pltpu.async_copy(
            don_so, don_out.at[pl.ds(ob, BATCH)], sem_g))
        out_copies.append(pltpu.async_copy(
            ter_so, ter_out.at[pl.ds(ob, BATCH)], sem_g))
        out_copies.append(pltpu.async_copy(
            tou_so, tou_out.at[pl.ds(ob, BATCH)], sem_g))
        out_copies.append(pltpu.async_copy(
            ens_so, ens_out.at[pl.ds(ob, BATCH)], sem_g))
        for cp in out_copies:
            cp.wait()
        return carry

    lax.fori_loop(0, EPW, env_body, 0)


_OUT_TYPE = (
    jax.ShapeDtypeStruct((N_ENV * BATCH // OBS_PR, 128), jnp.float32),
    jax.ShapeDtypeStruct((N_ENV * BATCH // OBS_PR, 128), jnp.float32),
    jax.ShapeDtypeStruct((N_ENV * BATCH // ACT_PR, 128), jnp.float32),
    jax.ShapeDtypeStruct((N_ENV * BATCH,), jnp.float32),
    jax.ShapeDtypeStruct((N_ENV * BATCH,), jnp.int32),
    jax.ShapeDtypeStruct((N_ENV * BATCH,), jnp.int32),
    jax.ShapeDtypeStruct((N_ENV * BATCH,), jnp.int32),
    jax.ShapeDtypeStruct((N_ENV * BATCH,), jnp.int32),
)

_SCRATCH = [
    pltpu.VMEM((BATCH,), jnp.int32),
    [pltpu.VMEM((CH,), jnp.int32)] * NCHUNK,
    [pltpu.VMEM((CH,), jnp.int32)] * NCHUNK,
    pltpu.VMEM((BATCH + L,), jnp.int32),
    pltpu.VMEM((BATCH + L,), jnp.int32),
    [pltpu.VMEM((CH, 128), jnp.float32)] * NCHUNK,
    [pltpu.VMEM((CH, 128), jnp.float32)] * NCHUNK,
    [pltpu.VMEM((CH, 128), jnp.float32)] * NCHUNK,
    pltpu.VMEM((CH // OBS_PR, 128), jnp.float32),
    pltpu.VMEM((CH // OBS_PR, 128), jnp.float32),
    pltpu.VMEM((CH // ACT_PR, 128), jnp.float32),
    pltpu.VMEM((BUF,), jnp.float32),
    pltpu.VMEM((BUF,), jnp.int32),
    pltpu.VMEM((BUF,), jnp.int32),
    pltpu.VMEM((BUF,), jnp.int32),
    pltpu.VMEM((BATCH,), jnp.float32),
    pltpu.VMEM((BATCH,), jnp.int32),
    pltpu.VMEM((BATCH,), jnp.int32),
    pltpu.VMEM((BATCH,), jnp.int32),
    pltpu.VMEM((BATCH,), jnp.int32),
    pltpu.VMEM((N_OBS,), jnp.float32),
    pltpu.VMEM((N_OBS,), jnp.float32),
    pltpu.VMEM((N_ACT,), jnp.float32),
    pltpu.VMEM((EPW + L,), jnp.float32),
    pltpu.VMEM((EPW + L,), jnp.int32),
    pltpu.VMEM((EPW + L,), jnp.int32),
    pltpu.VMEM((EPW + L,), jnp.int32),
    pltpu.VMEM((L,), jnp.int32),
    pltpu.SemaphoreType.DMA,
]

_sc_call = pl.kernel(
    _body,
    out_type=_OUT_TYPE,
    mesh=plsc.VectorSubcoreMesh(core_axis_name="c", subcore_axis_name="s",
                                num_cores=NC, num_subcores=NS),
    scratch_types=_SCRATCH,
    compiler_params=pltpu.CompilerParams(needs_layout_passes=False),
)


def kernel(observations_buf, next_observations_buf, actions_buf, rewards_buf,
           dones_buf, terminations_buf, time_outs_buf,
           obs, actions_in, rewards_in, next_obs,
           dones_in, terminations_in, time_outs_in,
           indices, ptr):
    p = jnp.asarray(ptr, jnp.int32) % BUF
    p_arr = jnp.full((L,), p, jnp.int32)
    zf = jnp.float32(0)
    obs_flat = observations_buf.reshape(N_ENV * BUF * N_OBS // 128, 128) + zf
    nobs_flat = (next_observations_buf.reshape(N_ENV * BUF * N_OBS // 128, 128)
                 + zf)
    act_flat = actions_buf.reshape(N_ENV * BUF * N_ACT // 128, 128) + zf
    idx1 = indices.reshape(-1)
    (obs128_o, nobs128_o, act128_o, rewards, dones, terminations, time_outs,
     ens) = _sc_call(
        obs_flat, nobs_flat, act_flat,
        rewards_buf.reshape(-1), dones_buf.reshape(-1),
        terminations_buf.reshape(-1), time_outs_buf.reshape(-1),
        obs.reshape(-1), next_obs.reshape(-1), actions_in.reshape(-1),
        rewards_in, dones_in, terminations_in, time_outs_in,
        idx1, p_arr)
    observations = obs128_o.reshape(N_ENV * BATCH, N_OBS) + zf
    next_observations = nobs128_o.reshape(N_ENV * BATCH, N_OBS) + zf
    actions = act128_o.reshape(N_ENV * BATCH, N_ACT) + zf
    return (observations, next_observations, actions, rewards, dones,
            terminations, time_outs, ens)


# speedup vs baseline: 1.7045x; 1.2071x over previous
import jax
import jax.numpy as jnp
from jax import lax
from jax.experimental import pallas as pl
from jax.experimental.pallas import tpu as pltpu
from jax.experimental.pallas import tpu_sc as plsc

N_ENV = 512
BUF = 1024
N_OBS = 64
N_ACT = 16
BATCH = 256

NC = 2
NS = 16
NW = NC * NS
L = 16
EPW = N_ENV // NW
NCHUNK = 2
CH = BATCH // NCHUNK
NG = BATCH // L


def _worker_id():
    return lax.axis_index("s") * NC + lax.axis_index("c")


def _body(obs_flat, nobs_flat, act_flat, rew_buf, don_buf, ter_buf, tou_buf,
          obs_new, nobs_new, act_new, rew_new, don_new, ter_new, tou_new,
          idx3, p_arr,
          obs_out, nobs_out, act_out, rew_out, don_out, ter_out, tou_out,
          ens_out,
          idx_v, gidx_a, gidx_b,
          obs_rows_a, obs_rows_b, nobs_rows_a, nobs_rows_b,
          act_rows_a, act_rows_b,
          rew_row, don_row, ter_row, tou_row,
          rew_so, don_so, ter_so, tou_so, ens_so,
          obs_ne, nobs_ne, act_ne,
          rew16, don16, ter16, tou16, p_v,
          sem_g, sem_s, sem_o):
    w = _worker_id()
    base_env = w * EPW

    pltpu.sync_copy(p_arr, p_v)
    pltpu.sync_copy(rew_new.at[pl.ds(base_env, EPW)], rew16.at[pl.ds(0, EPW)])
    pltpu.sync_copy(don_new.at[pl.ds(base_env, EPW)], don16.at[pl.ds(0, EPW)])
    pltpu.sync_copy(ter_new.at[pl.ds(base_env, EPW)], ter16.at[pl.ds(0, EPW)])
    pltpu.sync_copy(tou_new.at[pl.ds(base_env, EPW)], tou16.at[pl.ds(0, EPW)])
    pv = p_v[...]
    ones16 = jnp.ones((L,), jnp.int32)
    for g in range(NG):
        ens_so[pl.ds(g * L, L)] = ones16

    def env_body(j, carry):
        e = base_env + j
        pltpu.sync_copy(idx3.at[e], idx_v)

        ebase = e * BUF
        hit_acc = jnp.zeros((L,), jnp.bool_)
        gidx = (gidx_a, gidx_b)
        for g in range(NG):
            iv = idx_v[g // 8, pl.ds((g % 8) * L, L)]
            gidx[g // 8][pl.ds((g % 8) * L, L)] = iv + ebase
            hit_acc = jnp.logical_or(hit_acc, iv == pv)
        anyhit = plsc.all_reduce_population_count(hit_acc)[0] > 0

        @pl.when(j > 0)
        def _drain_prev():
            ob_p = e * BATCH
            for c in range(NCHUNK):
                pltpu.make_async_copy(
                    obs_rows_a, obs_out.at[pl.ds(ob_p, CH)], sem_o).wait()
                pltpu.make_async_copy(
                    nobs_rows_a, nobs_out.at[pl.ds(ob_p, CH)], sem_o).wait()
                pltpu.make_async_copy(
                    act_rows_a, act_out.at[pl.ds(ob_p, CH)], sem_o).wait()
            pltpu.make_async_copy(
                rew_so, rew_out.at[pl.ds(ob_p, BATCH)], sem_o).wait()
            pltpu.make_async_copy(
                don_so, don_out.at[pl.ds(ob_p, BATCH)], sem_o).wait()
            pltpu.make_async_copy(
                ter_so, ter_out.at[pl.ds(ob_p, BATCH)], sem_o).wait()
            pltpu.make_async_copy(
                tou_so, tou_out.at[pl.ds(ob_p, BATCH)], sem_o).wait()
            pltpu.make_async_copy(
                ens_so, ens_out.at[pl.ds(ob_p, BATCH)], sem_o).wait()

        obs_rows = (obs_rows_a, obs_rows_b)
        nobs_rows = (nobs_rows_a, nobs_rows_b)
        act_rows = (act_rows_a, act_rows_b)
        g_copies = []
        for c in range(NCHUNK):
            g_copies.append(pltpu.async_copy(
                obs_flat.at[gidx[c]], obs_rows[c], sem_g))
            g_copies.append(pltpu.async_copy(
                nobs_flat.at[gidx[c]], nobs_rows[c], sem_g))
            g_copies.append(pltpu.async_copy(
                act_flat.at[gidx[c]], act_rows[c], sem_g))
        s_copies = [
            pltpu.async_copy(rew_buf.at[e], rew_row, sem_s),
            pltpu.async_copy(don_buf.at[e], don_row, sem_s),
            pltpu.async_copy(ter_buf.at[e], ter_row, sem_s),
            pltpu.async_copy(tou_buf.at[e], tou_row, sem_s),
        ]

        @pl.when(anyhit)
        def _load_new():
            pltpu.sync_copy(obs_new.at[e], obs_ne)
            pltpu.sync_copy(nobs_new.at[e], nobs_ne)
            pltpu.sync_copy(act_new.at[e], act_ne)

        rew_e = jnp.full((L,), rew16[pl.ds(j, L)][0])
        don_e = jnp.full((L,), don16[pl.ds(j, L)][0])
        ter_e = jnp.full((L,), ter16[pl.ds(j, L)][0])
        tou_e = jnp.full((L,), tou16[pl.ds(j, L)][0])

        for cp in s_copies:
            cp.wait()
        for g in range(NG):
            iv = idx_v[g // 8, pl.ds((g % 8) * L, L)]
            m = iv == pv
            rew_so[pl.ds(g * L, L)] = jnp.where(
                m, rew_e, plsc.load_gather(rew_row, [iv]))
            don_so[pl.ds(g * L, L)] = jnp.where(
                m, don_e, plsc.load_gather(don_row, [iv]))
            ter_so[pl.ds(g * L, L)] = jnp.where(
                m, ter_e, plsc.load_gather(ter_row, [iv]))
            tou_so[pl.ds(g * L, L)] = jnp.where(
                m, tou_e, plsc.load_gather(tou_row, [iv]))

        for cp in g_copies:
            cp.wait()

        @pl.when(anyhit)
        def _fix():
            onew = [obs_ne[pl.ds(k * L, L)] for k in range(N_OBS // L)]
            nnew = [nobs_ne[pl.ds(k * L, L)] for k in range(N_OBS // L)]
            anew = act_ne[...]
            for g in range(NG):
                iv = idx_v[g // 8, pl.ds((g % 8) * L, L)]
                m = iv == pv
                mi = jnp.where(m, 1, 0).astype(jnp.int32)

                @pl.when(plsc.all_reduce_population_count(m)[0] > 0)
                def _fix_group(g=g, mi=mi):
                    ck = g // 8
                    for lane in range(L):
                        @pl.when(mi[lane] != 0)
                        def _fix_lane(g=g, lane=lane, ck=ck):
                            b = (g % 8) * L + lane
                            for k in range(N_OBS // L):
                                obs_rows[ck][b, pl.ds(k * L, L)] = onew[k]
                                nobs_rows[ck][b, pl.ds(k * L, L)] = nnew[k]
                            act_rows[ck][b, :] = anew

        ob = e * BATCH
        o_copies = []
        for c in range(NCHUNK):
            o_copies.append(pltpu.async_copy(
                obs_rows[c], obs_out.at[pl.ds(ob + c * CH, CH)], sem_o))
            o_copies.append(pltpu.async_copy(
                nobs_rows[c], nobs_out.at[pl.ds(ob + c * CH, CH)], sem_o))
            o_copies.append(pltpu.async_copy(
                act_rows[c], act_out.at[pl.ds(ob + c * CH, CH)], sem_o))
        o_copies.append(pltpu.async_copy(
            rew_so, rew_out.at[pl.ds(ob, BATCH)], sem_o))
        o_copies.append(pltpu.async_copy(
            don_so, don_out.at[pl.ds(ob, BATCH)], sem_o))
        o_copies.append(pltpu.async_copy(
            ter_so, ter_out.at[pl.ds(ob, BATCH)], sem_o))
        o_copies.append(pltpu.async_copy(
            tou_so, tou_out.at[pl.ds(ob, BATCH)], sem_o))
        o_copies.append(pltpu.async_copy(
            ens_so, ens_out.at[pl.ds(ob, BATCH)], sem_o))
        return carry

    lax.fori_loop(0, EPW, env_body, 0)

    e_last = base_env + EPW - 1
    ob_l = e_last * BATCH
    for c in range(NCHUNK):
        pltpu.make_async_copy(
            obs_rows_a, obs_out.at[pl.ds(ob_l, CH)], sem_o).wait()
        pltpu.make_async_copy(
            nobs_rows_a, nobs_out.at[pl.ds(ob_l, CH)], sem_o).wait()
        pltpu.make_async_copy(
            act_rows_a, act_out.at[pl.ds(ob_l, CH)], sem_o).wait()
    pltpu.make_async_copy(
        rew_so, rew_out.at[pl.ds(ob_l, BATCH)], sem_o).wait()
    pltpu.make_async_copy(
        don_so, don_out.at[pl.ds(ob_l, BATCH)], sem_o).wait()
    pltpu.make_async_copy(
        ter_so, ter_out.at[pl.ds(ob_l, BATCH)], sem_o).wait()
    pltpu.make_async_copy(
        tou_so, tou_out.at[pl.ds(ob_l, BATCH)], sem_o).wait()
    pltpu.make_async_copy(
        ens_so, ens_out.at[pl.ds(ob_l, BATCH)], sem_o).wait()


_OUT_TYPE = (
    jax.ShapeDtypeStruct((N_ENV * BATCH, N_OBS), jnp.float32),
    jax.ShapeDtypeStruct((N_ENV * BATCH, N_OBS), jnp.float32),
    jax.ShapeDtypeStruct((N_ENV * BATCH, N_ACT), jnp.float32),
    jax.ShapeDtypeStruct((N_ENV * BATCH,), jnp.float32),
    jax.ShapeDtypeStruct((N_ENV * BATCH,), jnp.int32),
    jax.ShapeDtypeStruct((N_ENV * BATCH,), jnp.int32),
    jax.ShapeDtypeStruct((N_ENV * BATCH,), jnp.int32),
    jax.ShapeDtypeStruct((N_ENV * BATCH,), jnp.int32),
)

_SCRATCH = [
    pltpu.VMEM((NCHUNK, CH), jnp.int32),
    pltpu.VMEM((CH,), jnp.int32),
    pltpu.VMEM((CH,), jnp.int32),
    pltpu.VMEM((CH, N_OBS), jnp.float32),
    pltpu.VMEM((CH, N_OBS), jnp.float32),
    pltpu.VMEM((CH, N_OBS), jnp.float32),
    pltpu.VMEM((CH, N_OBS), jnp.float32),
    pltpu.VMEM((CH, N_ACT), jnp.float32),
    pltpu.VMEM((CH, N_ACT), jnp.float32),
    pltpu.VMEM((BUF,), jnp.float32),
    pltpu.VMEM((BUF,), jnp.int32),
    pltpu.VMEM((BUF,), jnp.int32),
    pltpu.VMEM((BUF,), jnp.int32),
    pltpu.VMEM((BATCH,), jnp.float32),
    pltpu.VMEM((BATCH,), jnp.int32),
    pltpu.VMEM((BATCH,), jnp.int32),
    pltpu.VMEM((BATCH,), jnp.int32),
    pltpu.VMEM((BATCH,), jnp.int32),
    pltpu.VMEM((N_OBS,), jnp.float32),
    pltpu.VMEM((N_OBS,), jnp.float32),
    pltpu.VMEM((N_ACT,), jnp.float32),
    pltpu.VMEM((EPW + L,), jnp.float32),
    pltpu.VMEM((EPW + L,), jnp.int32),
    pltpu.VMEM((EPW + L,), jnp.int32),
    pltpu.VMEM((EPW + L,), jnp.int32),
    pltpu.VMEM((L,), jnp.int32),
    pltpu.SemaphoreType.DMA,
    pltpu.SemaphoreType.DMA,
    pltpu.SemaphoreType.DMA,
]

_sc_call = pl.kernel(
    _body,
    out_type=_OUT_TYPE,
    mesh=plsc.VectorSubcoreMesh(core_axis_name="c", subcore_axis_name="s",
                                num_cores=NC, num_subcores=NS),
    scratch_types=_SCRATCH,
    compiler_params=pltpu.CompilerParams(needs_layout_passes=False,
                                         use_tc_tiling_on_sc=False),
)


def kernel(observations_buf, next_observations_buf, actions_buf, rewards_buf,
           dones_buf, terminations_buf, time_outs_buf,
           obs, actions_in, rewards_in, next_obs,
           dones_in, terminations_in, time_outs_in,
           indices, ptr):
    p = jnp.asarray(ptr, jnp.int32) % BUF
    p_arr = jnp.full((L,), p, jnp.int32)
    obs_flat = observations_buf.reshape(N_ENV * BUF, N_OBS)
    nobs_flat = next_observations_buf.reshape(N_ENV * BUF, N_OBS)
    act_flat = actions_buf.reshape(N_ENV * BUF, N_ACT)
    idx3 = indices.reshape(N_ENV, NCHUNK, CH)
    return _sc_call(
        obs_flat, nobs_flat, act_flat, rewards_buf, dones_buf,
        terminations_buf, time_outs_buf,
        obs, next_obs, actions_in, rewards_in,
        dones_in, terminations_in, time_outs_in,
        idx3, p_arr)


# speedup vs baseline: 1.7069x; 1.0014x over previous
import jax
import jax.numpy as jnp
from jax import lax
from jax.experimental import pallas as pl
from jax.experimental.pallas import tpu as pltpu
from jax.experimental.pallas import tpu_sc as plsc

N_ENV = 512
BUF = 1024
N_OBS = 64
N_ACT = 16
BATCH = 256

NC = 2
NS = 16
NW = NC * NS
L = 16
EPW = N_ENV // NW
NCHUNK = 2
CH = BATCH // NCHUNK
NG = BATCH // L


def _worker_id():
    return lax.axis_index("s") * NC + lax.axis_index("c")


def _body(obs_flat, nobs_flat, act_flat, rew_buf, don_buf, ter_buf, tou_buf,
          obs_new, nobs_new, act_new, rew_new, don_new, ter_new, tou_new,
          idx3, p_arr,
          obs_out, nobs_out, act_out, rew_out, don_out, ter_out, tou_out,
          ens_out,
          idx_v, gidx_a, gidx_b,
          obs_rows_a, obs_rows_b, nobs_rows_a, nobs_rows_b,
          act_rows_a, act_rows_b,
          rew_row, don_row, ter_row, tou_row,
          rew_so, don_so, ter_so, tou_so, ens_so,
          obs_ne, nobs_ne, act_ne,
          rew16, don16, ter16, tou16, p_v,
          sem_g, sem_s, sem_o):
    w = _worker_id()
    base_env = w * EPW

    pltpu.sync_copy(p_arr, p_v)
    pltpu.sync_copy(rew_new.at[pl.ds(base_env, EPW)], rew16.at[pl.ds(0, EPW)])
    pltpu.sync_copy(don_new.at[pl.ds(base_env, EPW)], don16.at[pl.ds(0, EPW)])
    pltpu.sync_copy(ter_new.at[pl.ds(base_env, EPW)], ter16.at[pl.ds(0, EPW)])
    pltpu.sync_copy(tou_new.at[pl.ds(base_env, EPW)], tou16.at[pl.ds(0, EPW)])
    pv = p_v[...]
    ones16 = jnp.ones((L,), jnp.int32)
    for g in range(NG):
        ens_so[pl.ds(g * L, L)] = ones16

    def env_body(j, carry):
        e = base_env + j
        pltpu.sync_copy(idx3.at[e], idx_v)

        ebase = e * BUF
        hit_acc = jnp.zeros((L,), jnp.bool_)
        gidx = (gidx_a, gidx_b)
        for g in range(NG):
            iv = idx_v[g // 8, pl.ds((g % 8) * L, L)]
            gidx[g // 8][pl.ds((g % 8) * L, L)] = iv + ebase
            hit_acc = jnp.logical_or(hit_acc, iv == pv)
        anyhit = plsc.all_reduce_population_count(hit_acc)[0] > 0

        @pl.when(j > 0)
        def _drain_prev():
            ob_p = e * BATCH
            for c in range(NCHUNK):
                pltpu.make_async_copy(
                    obs_rows_a, obs_out.at[pl.ds(ob_p, CH)], sem_o).wait()
                pltpu.make_async_copy(
                    nobs_rows_a, nobs_out.at[pl.ds(ob_p, CH)], sem_o).wait()
                pltpu.make_async_copy(
                    act_rows_a, act_out.at[pl.ds(ob_p, CH)], sem_o).wait()
            pltpu.make_async_copy(
                rew_so, rew_out.at[pl.ds(ob_p, BATCH)], sem_o).wait()
            pltpu.make_async_copy(
                don_so, don_out.at[pl.ds(ob_p, BATCH)], sem_o).wait()
            pltpu.make_async_copy(
                ter_so, ter_out.at[pl.ds(ob_p, BATCH)], sem_o).wait()
            pltpu.make_async_copy(
                tou_so, tou_out.at[pl.ds(ob_p, BATCH)], sem_o).wait()
            pltpu.make_async_copy(
                ens_so, ens_out.at[pl.ds(ob_p, BATCH)], sem_o).wait()

        obs_rows = (obs_rows_a, obs_rows_b)
        nobs_rows = (nobs_rows_a, nobs_rows_b)
        act_rows = (act_rows_a, act_rows_b)
        g_copies = []
        for c in range(NCHUNK):
            g_copies.append(pltpu.async_copy(
                obs_flat.at[gidx[c]], obs_rows[c], sem_g))
            g_copies.append(pltpu.async_copy(
                nobs_flat.at[gidx[c]], nobs_rows[c], sem_g))
            g_copies.append(pltpu.async_copy(
                act_flat.at[gidx[c]], act_rows[c], sem_g))
        s_copies = [
            pltpu.async_copy(rew_buf.at[e], rew_row, sem_s),
            pltpu.async_copy(don_buf.at[e], don_row, sem_s),
            pltpu.async_copy(ter_buf.at[e], ter_row, sem_s),
            pltpu.async_copy(tou_buf.at[e], tou_row, sem_s),
        ]

        @pl.when(anyhit)
        def _load_new():
            pltpu.sync_copy(obs_new.at[e], obs_ne)
            pltpu.sync_copy(nobs_new.at[e], nobs_ne)
            pltpu.sync_copy(act_new.at[e], act_ne)

        rew_e = jnp.full((L,), rew16[pl.ds(j, L)][0])
        don_e = jnp.full((L,), don16[pl.ds(j, L)][0])
        ter_e = jnp.full((L,), ter16[pl.ds(j, L)][0])
        tou_e = jnp.full((L,), tou16[pl.ds(j, L)][0])

        for cp in s_copies:
            cp.wait()
        for g in range(NG):
            iv = idx_v[g // 8, pl.ds((g % 8) * L, L)]
            m = iv == pv
            rew_so[pl.ds(g * L, L)] = jnp.where(
                m, rew_e, plsc.load_gather(rew_row, [iv]))
            don_so[pl.ds(g * L, L)] = jnp.where(
                m, don_e, plsc.load_gather(don_row, [iv]))
            ter_so[pl.ds(g * L, L)] = jnp.where(
                m, ter_e, plsc.load_gather(ter_row, [iv]))
            tou_so[pl.ds(g * L, L)] = jnp.where(
                m, tou_e, plsc.load_gather(tou_row, [iv]))

        for cp in g_copies:
            cp.wait()

        @pl.when(anyhit)
        def _fix():
            onew = [obs_ne[pl.ds(k * L, L)] for k in range(N_OBS // L)]
            nnew = [nobs_ne[pl.ds(k * L, L)] for k in range(N_OBS // L)]
            anew = act_ne[...]
            for g in range(NG):
                iv = idx_v[g // 8, pl.ds((g % 8) * L, L)]
                m = iv == pv
                mi = jnp.where(m, 1, 0).astype(jnp.int32)

                @pl.when(plsc.all_reduce_population_count(m)[0] > 0)
                def _fix_group(g=g, mi=mi):
                    ck = g // 8
                    for lane in range(L):
                        @pl.when(mi[lane] != 0)
                        def _fix_lane(g=g, lane=lane, ck=ck):
                            b = (g % 8) * L + lane
                            for k in range(N_OBS // L):
                                obs_rows[ck][b, pl.ds(k * L, L)] = onew[k]
                                nobs_rows[ck][b, pl.ds(k * L, L)] = nnew[k]
                            act_rows[ck][b, :] = anew

        ob = e * BATCH
        o_copies = []
        for c in range(NCHUNK):
            o_copies.append(pltpu.async_copy(
                obs_rows[c], obs_out.at[pl.ds(ob + c * CH, CH)], sem_o))
            o_copies.append(pltpu.async_copy(
                nobs_rows[c], nobs_out.at[pl.ds(ob + c * CH, CH)], sem_o))
            o_copies.append(pltpu.async_copy(
                act_rows[c], act_out.at[pl.ds(ob + c * CH, CH)], sem_o))
        o_copies.append(pltpu.async_copy(
            rew_so, rew_out.at[pl.ds(ob, BATCH)], sem_o))
        o_copies.append(pltpu.async_copy(
            don_so, don_out.at[pl.ds(ob, BATCH)], sem_o))
        o_copies.append(pltpu.async_copy(
            ter_so, ter_out.at[pl.ds(ob, BATCH)], sem_o))
        o_copies.append(pltpu.async_copy(
            tou_so, tou_out.at[pl.ds(ob, BATCH)], sem_o))
        o_copies.append(pltpu.async_copy(
            ens_so, ens_out.at[pl.ds(ob, BATCH)], sem_o))
        return carry

    lax.fori_loop(0, EPW, env_body, 0)

    e_last = base_env + EPW - 1
    ob_l = e_last * BATCH
    for c in range(NCHUNK):
        pltpu.make_async_copy(
            obs_rows_a, obs_out.at[pl.ds(ob_l, CH)], sem_o).wait()
        pltpu.make_async_copy(
            nobs_rows_a, nobs_out.at[pl.ds(ob_l, CH)], sem_o).wait()
        pltpu.make_async_copy(
            act_rows_a, act_out.at[pl.ds(ob_l, CH)], sem_o).wait()
    pltpu.make_async_copy(
        rew_so, rew_out.at[pl.ds(ob_l, BATCH)], sem_o).wait()
    pltpu.make_async_copy(
        don_so, don_out.at[pl.ds(ob_l, BATCH)], sem_o).wait()
    pltpu.make_async_copy(
        ter_so, ter_out.at[pl.ds(ob_l, BATCH)], sem_o).wait()
    pltpu.make_async_copy(
        tou_so, tou_out.at[pl.ds(ob_l, BATCH)], sem_o).wait()
    pltpu.make_async_copy(
        ens_so, ens_out.at[pl.ds(ob_l, BATCH)], sem_o).wait()


_OUT_TYPE = (
    jax.ShapeDtypeStruct((N_ENV * BATCH, N_OBS), jnp.float32),
    jax.ShapeDtypeStruct((N_ENV * BATCH, N_OBS), jnp.float32),
    jax.ShapeDtypeStruct((N_ENV * BATCH, N_ACT), jnp.float32),
    jax.ShapeDtypeStruct((N_ENV * BATCH,), jnp.float32),
    jax.ShapeDtypeStruct((N_ENV * BATCH,), jnp.int32),
    jax.ShapeDtypeStruct((N_ENV * BATCH,), jnp.int32),
    jax.ShapeDtypeStruct((N_ENV * BATCH,), jnp.int32),
    jax.ShapeDtypeStruct((N_ENV * BATCH,), jnp.int32),
)

_SCRATCH = [
    pltpu.VMEM((NCHUNK, CH), jnp.int32),
    pltpu.VMEM((CH,), jnp.int32),
    pltpu.VMEM((CH,), jnp.int32),
    pltpu.VMEM((CH, N_OBS), jnp.float32),
    pltpu.VMEM((CH, N_OBS), jnp.float32),
    pltpu.VMEM((CH, N_OBS), jnp.float32),
    pltpu.VMEM((CH, N_OBS), jnp.float32),
    pltpu.VMEM((CH, N_ACT), jnp.float32),
    pltpu.VMEM((CH, N_ACT), jnp.float32),
    pltpu.VMEM((BUF,), jnp.float32),
    pltpu.VMEM((BUF,), jnp.int32),
    pltpu.VMEM((BUF,), jnp.int32),
    pltpu.VMEM((BUF,), jnp.int32),
    pltpu.VMEM((BATCH,), jnp.float32),
    pltpu.VMEM((BATCH,), jnp.int32),
    pltpu.VMEM((BATCH,), jnp.int32),
    pltpu.VMEM((BATCH,), jnp.int32),
    pltpu.VMEM((BATCH,), jnp.int32),
    pltpu.VMEM((N_OBS,), jnp.float32),
    pltpu.VMEM((N_OBS,), jnp.float32),
    pltpu.VMEM((N_ACT,), jnp.float32),
    pltpu.VMEM((EPW + L,), jnp.float32),
    pltpu.VMEM((EPW + L,), jnp.int32),
    pltpu.VMEM((EPW + L,), jnp.int32),
    pltpu.VMEM((EPW + L,), jnp.int32),
    pltpu.VMEM((L,), jnp.int32),
    pltpu.SemaphoreType.DMA,
    pltpu.SemaphoreType.DMA,
    pltpu.SemaphoreType.DMA,
]

_sc_call = pl.kernel(
    _body,
    out_type=_OUT_TYPE,
    mesh=plsc.VectorSubcoreMesh(core_axis_name="c", subcore_axis_name="s",
                                num_cores=NC, num_subcores=NS),
    scratch_types=_SCRATCH,
    compiler_params=pltpu.CompilerParams(needs_layout_passes=False,
                                         use_tc_tiling_on_sc=False),
)


def kernel(observations_buf, next_observations_buf, actions_buf, rewards_buf,
           dones_buf, terminations_buf, time_outs_buf,
           obs, actions_in, rewards_in, next_obs,
           dones_in, terminations_in, time_outs_in,
           indices, ptr):
    p = jnp.asarray(ptr, jnp.int32) % BUF
    p_arr = jnp.full((L,), p, jnp.int32)
    obs_flat = lax.optimization_barrier(
        observations_buf.reshape(N_ENV * BUF, N_OBS))
    nobs_flat = lax.optimization_barrier(
        next_observations_buf.reshape(N_ENV * BUF, N_OBS))
    act_flat = lax.optimization_barrier(
        actions_buf.reshape(N_ENV * BUF, N_ACT))
    idx3 = indices.reshape(N_ENV, NCHUNK, CH)
    return _sc_call(
        obs_flat, nobs_flat, act_flat, rewards_buf, dones_buf,
        terminations_buf, time_outs_buf,
        obs, next_obs, actions_in, rewards_in,
        dones_in, terminations_in, time_outs_in,
        idx3, p_arr)
